# baseline (speedup 1.0000x reference)
import functools

import jax
import jax.numpy as jnp
from jax import lax
from jax.experimental import pallas as pl
from jax.experimental.pallas import tpu as pltpu
from jax.experimental.pallas import tpu_sc as plsc

N_USERS = 6000
D = 128
NC = 2
NS = 16
L = 16
NW = NC * NS
CHUNK = 64
U = 16
BN = 1024


def _mm2_body(x_ref, wl_ref, wr_ref, xl_ref, xr_ref):
  x = x_ref[...]
  xl_ref[...] = jnp.dot(x, wl_ref[...], preferred_element_type=jnp.float32)
  xr_ref[...] = jnp.dot(x, wr_ref[...], preferred_element_type=jnp.float32)


def _mm2(x, wl, wr):
  n = x.shape[0]
  return pl.pallas_call(
      _mm2_body,
      grid=(n // BN,),
      in_specs=[
          pl.BlockSpec((BN, D), lambda i: (i, 0)),
          pl.BlockSpec((D, D), lambda i: (0, 0)),
          pl.BlockSpec((D, D), lambda i: (0, 0)),
      ],
      out_specs=[
          pl.BlockSpec((BN, D), lambda i: (i, 0)),
          pl.BlockSpec((BN, D), lambda i: (i, 0)),
      ],
      out_shape=[jax.ShapeDtypeStruct((n, D), jnp.float32)] * 2,
  )(x, wl, wr)


def _comb_mm2_body(acc_ref, s_ref, b_ref, res_ref, wl_ref, wr_ref,
                   h_ref, xl_ref, xr_ref):
  s = jnp.sum(s_ref[...], axis=0) + 1e-16
  h = (acc_ref[0] + acc_ref[1]) / s[:, None] + b_ref[...] + res_ref[...]
  h = jnp.maximum(h, 0.0)
  h_ref[...] = h
  xl_ref[...] = jnp.dot(h, wl_ref[...], preferred_element_type=jnp.float32)
  xr_ref[...] = jnp.dot(h, wr_ref[...], preferred_element_type=jnp.float32)


def _comb_mm2(acc, s, b, res, wl, wr):
  n = res.shape[0]
  nw = s.shape[0]
  return pl.pallas_call(
      _comb_mm2_body,
      grid=(n // BN,),
      in_specs=[
          pl.BlockSpec((NC, BN, D), lambda i: (0, i, 0)),
          pl.BlockSpec((nw, BN), lambda i: (0, i)),
          pl.BlockSpec((1, D), lambda i: (0, 0)),
          pl.BlockSpec((BN, D), lambda i: (i, 0)),
          pl.BlockSpec((D, D), lambda i: (0, 0)),
          pl.BlockSpec((D, D), lambda i: (0, 0)),
      ],
      out_specs=[
          pl.BlockSpec((BN, D), lambda i: (i, 0)),
          pl.BlockSpec((BN, D), lambda i: (i, 0)),
          pl.BlockSpec((BN, D), lambda i: (i, 0)),
      ],
      out_shape=[jax.ShapeDtypeStruct((n, D), jnp.float32)] * 3,
  )(acc, s, b, res, wl, wr)


def _final_body(acc_ref, s_ref, b_ref, res_ref, y_ref):
  s = jnp.sum(s_ref[...], axis=0) + 1e-16
  y_ref[...] = (acc_ref[0] + acc_ref[1]) / s[:, None] + b_ref[...] + res_ref[...]


def _final(acc, s, b, res):
  n = res.shape[0]
  nw = s.shape[0]
  return pl.pallas_call(
      _final_body,
      grid=(n // BN,),
      in_specs=[
          pl.BlockSpec((NC, BN, D), lambda i: (0, i, 0)),
          pl.BlockSpec((nw, BN), lambda i: (0, i)),
          pl.BlockSpec((1, D), lambda i: (0, 0)),
          pl.BlockSpec((BN, D), lambda i: (i, 0)),
      ],
      out_specs=pl.BlockSpec((BN, D), lambda i: (i, 0)),
      out_shape=jax.ShapeDtypeStruct((n, D), jnp.float32),
  )(acc, s, b, res)


def _sc_mesh():
  return plsc.VectorSubcoreMesh(
      core_axis_name="c", subcore_axis_name="s", num_cores=NC, num_subcores=NS)


_SC_PARAMS = pltpu.CompilerParams(needs_layout_passes=False)


@functools.lru_cache(maxsize=None)
def _make_sc_edge(e_pad, n_pad, per_w):
  n_chunks = per_w // CHUNK
  n_quads = n_chunks // 4
  rpt = n_pad // NS

  @functools.partial(
      pl.kernel,
      out_type=[
          jax.ShapeDtypeStruct((NW, n_pad), jnp.float32),
          jax.ShapeDtypeStruct((NC, n_pad, D), jnp.float32),
      ],
      mesh=_sc_mesh(),
      compiler_params=_SC_PARAMS,
      scratch_types=[
          pltpu.VMEM((2, CHUNK), jnp.int32),
          pltpu.VMEM((2, CHUNK), jnp.int32),
          pltpu.VMEM((2, CHUNK), jnp.int32),
          pltpu.VMEM((2, CHUNK), jnp.int32),
          pltpu.VMEM((CHUNK, D), jnp.float32),
          pltpu.VMEM((CHUNK, D), jnp.float32),
          pltpu.VMEM((CHUNK, D), jnp.float32),
          pltpu.VMEM((CHUNK, D), jnp.float32),
          pltpu.VMEM((D,), jnp.float32),
          pltpu.VMEM((n_pad,), jnp.float32),
          pltpu.VMEM_SHARED((n_pad, D), jnp.float32),
          pltpu.SemaphoreType.DMA,
          pltpu.SemaphoreType.DMA,
          pltpu.SemaphoreType.DMA,
          pltpu.SemaphoreType.DMA,
          pltpu.SemaphoreType.DMA,
          pltpu.SemaphoreType.DMA,
      ],
  )
  def sc_edge(xl_hbm, xr_hbm, idx_hbm, att_hbm,
              zero_nd_hbm,
              s_hbm, out_hbm,
              idx_0, idx_1, idx_2, idx_3, xlr_a, xrr_a, xlr_b, xrr_b,
              att_v, s_v, acc_sh,
              sem_a, sem_b, sem_i0, sem_i1, sem_i2, sem_i3):
    cid = lax.axis_index("c")
    sid = lax.axis_index("s")
    wid = cid * NS + sid
    pltpu.sync_copy(att_hbm, att_v)
    pltpu.sync_copy(zero_nd_hbm.at[pl.ds(sid * rpt, rpt)],
                    acc_sh.at[pl.ds(sid * rpt, rpt)])

    def zero_body(i, carry):
      s_v[pl.ds(i * L, L)] = jnp.zeros((L,), jnp.float32)
      return carry

    lax.fori_loop(0, n_pad // L, zero_body, 0)
    plsc.subcore_barrier()
    row16 = lax.iota(jnp.int32, L)
    att_q = [att_v[pl.ds(q * L, L)] for q in range(D // L)]

    ibufs = (idx_0, idx_1, idx_2, idx_3)
    isems = (sem_i0, sem_i1, sem_i2, sem_i3)
    RA = (xlr_a, xrr_a, sem_a)
    RB = (xlr_b, xrr_b, sem_b)

    def idx_issue(ci, k):
      pltpu.async_copy(idx_hbm.at[wid * n_chunks + ci], ibufs[k], isems[k])

    def idx_wait(ci, k):
      pltpu.make_async_copy(
          idx_hbm.at[wid * n_chunks + ci], ibufs[k], isems[k]).wait()

    def rows_issue(k, rbuf):
      idxb = ibufs[k]
      xlb, xrb, sem = rbuf
      pltpu.async_copy(xl_hbm.at[idxb.at[0]], xlb, sem)
      pltpu.async_copy(xr_hbm.at[idxb.at[1]], xrb, sem)

    def rows_wait(k, rbuf):
      idxb = ibufs[k]
      xlb, xrb, sem = rbuf
      pltpu.make_async_copy(xl_hbm.at[idxb.at[0]], xlb, sem).wait()
      pltpu.make_async_copy(xr_hbm.at[idxb.at[1]], xrb, sem).wait()

    def process(k, rbuf):
      idxb = ibufs[k]
      xlb, xrb, _ = rbuf
      dstb = idxb.at[1]

      def group_body(g, carry2):
        p16 = jnp.zeros((L,), jnp.float32)
        for u in range(L):
          j = g * L + u
          acc = None
          xs = []
          for q in range(D // L):
            a = xlb[j, pl.ds(q * L, L)]
            xs.append(a)
            t = a + xrb[j, pl.ds(q * L, L)]
            t = jnp.maximum(t, 0.2 * t)
            pr = att_q[q] * t
            acc = pr if acc is None else acc + pr
          pb = jnp.exp(jnp.full((L,), jnp.sum(acc), jnp.float32))
          p16 = jnp.where(row16 == u, pb, p16)
          for q in range(D // L):
            xlb[j, pl.ds(q * L, L)] = xs[q] * pb
        plsc.addupdate_scatter(s_v, [idxb[1, pl.ds(g * L, L)]], p16)
        return carry2

      lax.fori_loop(0, CHUNK // L, group_body, 0)
      pltpu.sync_copy(xlb, acc_sh.at[dstb], add=True)

    idx_issue(0, 0)
    idx_issue(1, 1)
    idx_wait(0, 0)
    rows_issue(0, RA)

    def quad_body(j, carry):
      c = 4 * j
      idx_issue(c + 2, 2)
      idx_issue(c + 3, 3)
      idx_wait(c + 1, 1)
      rows_issue(1, RB)
      rows_wait(0, RA)
      process(0, RA)
      idx_wait(c + 2, 2)
      rows_issue(2, RA)
      rows_wait(1, RB)
      process(1, RB)
      idx_wait(c + 3, 3)
      rows_issue(3, RB)
      rows_wait(2, RA)
      process(2, RA)

      @pl.when(j < n_quads - 1)
      def _():
        idx_issue(c + 4, 0)
        idx_issue(c + 5, 1)
        idx_wait(c + 4, 0)
        rows_issue(0, RA)

      rows_wait(3, RB)
      process(3, RB)
      return carry

    lax.fori_loop(0, n_quads, quad_body, 0)
    pltpu.sync_copy(s_v, s_hbm.at[wid])
    plsc.subcore_barrier()
    pltpu.sync_copy(acc_sh.at[pl.ds(sid * rpt, rpt)],
                    out_hbm.at[cid, pl.ds(sid * rpt, rpt)])

  return sc_edge


def kernel(edge_index, emb, Wl1, Wr1, att1, b1, Wl2, Wr2, att2, b2):
  n = emb.shape[0]
  e2 = edge_index.shape[1] + n
  n_pad = ((n + BN) // BN) * BN
  per_w = -(-e2 // (NW * 4 * CHUNK)) * 4 * CHUNK
  e_pad = per_w * NW

  loop = jnp.arange(n, dtype=jnp.int32)
  pad_e = e_pad - e2
  src = jnp.concatenate(
      [edge_index[0], loop, jnp.zeros((pad_e,), jnp.int32)])
  dst = jnp.concatenate(
      [edge_index[1], loop, jnp.full((pad_e,), n, jnp.int32)])
  idx_pk = jnp.stack([src.reshape(-1, CHUNK), dst.reshape(-1, CHUNK)], axis=1)
  emb_p = jnp.pad(emb, ((0, n_pad - n), (0, 0)))
  zero_nd = jnp.zeros((n_pad, D), jnp.float32)
  b1r = b1.reshape(1, D)
  b2r = b2.reshape(1, D)
  sc_edge = _make_sc_edge(e_pad, n_pad, per_w)

  xl1, xr1 = _mm2(emb_p, Wl1, Wr1)
  s1, acc1 = sc_edge(xl1, xr1, idx_pk, att1, zero_nd)
  h, xl2, xr2 = _comb_mm2(acc1, s1, b1r, emb_p, Wl2, Wr2)
  s2, acc2 = sc_edge(xl2, xr2, idx_pk, att2, zero_nd)
  y = _final(acc2, s2, b2r, h)

  y = y[:n]
  return (y[:N_USERS], y[N_USERS:])


# speedup vs baseline: 1.2996x; 1.2996x over previous
import functools

import jax
import jax.numpy as jnp
from jax import lax
from jax.experimental import pallas as pl
from jax.experimental.pallas import tpu as pltpu
from jax.experimental.pallas import tpu_sc as plsc

N_USERS = 6000
D = 128
NC = 2
NS = 16
L = 16
NW = NC * NS
CHUNK = 64
U = 16
BN = 1024


def _mm2_body(x_ref, wl_ref, wr_ref, xl_ref, xr_ref):
  x = x_ref[...]
  xl_ref[...] = jnp.dot(x, wl_ref[...], preferred_element_type=jnp.float32)
  xr_ref[...] = jnp.dot(x, wr_ref[...], preferred_element_type=jnp.float32)


def _mm2(x, wl, wr):
  n = x.shape[0]
  return pl.pallas_call(
      _mm2_body,
      grid=(n // BN,),
      in_specs=[
          pl.BlockSpec((BN, D), lambda i: (i, 0)),
          pl.BlockSpec((D, D), lambda i: (0, 0)),
          pl.BlockSpec((D, D), lambda i: (0, 0)),
      ],
      out_specs=[
          pl.BlockSpec((BN, D), lambda i: (i, 0)),
          pl.BlockSpec((BN, D), lambda i: (i, 0)),
      ],
      out_shape=[jax.ShapeDtypeStruct((n, D), jnp.float32)] * 2,
  )(x, wl, wr)


def _comb_mm2_body(acc_ref, s_ref, b_ref, res_ref, wl_ref, wr_ref,
                   h_ref, xl_ref, xr_ref):
  s = jnp.sum(s_ref[...], axis=0) + 1e-16
  h = (acc_ref[0] + acc_ref[1]) / s[:, None] + b_ref[...] + res_ref[...]
  h = jnp.maximum(h, 0.0)
  h_ref[...] = h
  xl_ref[...] = jnp.dot(h, wl_ref[...], preferred_element_type=jnp.float32)
  xr_ref[...] = jnp.dot(h, wr_ref[...], preferred_element_type=jnp.float32)


def _comb_mm2(acc, s, b, res, wl, wr):
  n = res.shape[0]
  nw = s.shape[0]
  return pl.pallas_call(
      _comb_mm2_body,
      grid=(n // BN,),
      in_specs=[
          pl.BlockSpec((NC, BN, D), lambda i: (0, i, 0)),
          pl.BlockSpec((nw, BN), lambda i: (0, i)),
          pl.BlockSpec((1, D), lambda i: (0, 0)),
          pl.BlockSpec((BN, D), lambda i: (i, 0)),
          pl.BlockSpec((D, D), lambda i: (0, 0)),
          pl.BlockSpec((D, D), lambda i: (0, 0)),
      ],
      out_specs=[
          pl.BlockSpec((BN, D), lambda i: (i, 0)),
          pl.BlockSpec((BN, D), lambda i: (i, 0)),
          pl.BlockSpec((BN, D), lambda i: (i, 0)),
      ],
      out_shape=[jax.ShapeDtypeStruct((n, D), jnp.float32)] * 3,
  )(acc, s, b, res, wl, wr)


def _final_body(acc_ref, s_ref, b_ref, res_ref, y_ref):
  s = jnp.sum(s_ref[...], axis=0) + 1e-16
  y_ref[...] = (acc_ref[0] + acc_ref[1]) / s[:, None] + b_ref[...] + res_ref[...]


def _final(acc, s, b, res):
  n = res.shape[0]
  nw = s.shape[0]
  return pl.pallas_call(
      _final_body,
      grid=(n // BN,),
      in_specs=[
          pl.BlockSpec((NC, BN, D), lambda i: (0, i, 0)),
          pl.BlockSpec((nw, BN), lambda i: (0, i)),
          pl.BlockSpec((1, D), lambda i: (0, 0)),
          pl.BlockSpec((BN, D), lambda i: (i, 0)),
      ],
      out_specs=pl.BlockSpec((BN, D), lambda i: (i, 0)),
      out_shape=jax.ShapeDtypeStruct((n, D), jnp.float32),
  )(acc, s, b, res)


def _sc_mesh():
  return plsc.VectorSubcoreMesh(
      core_axis_name="c", subcore_axis_name="s", num_cores=NC, num_subcores=NS)


_SC_PARAMS = pltpu.CompilerParams(needs_layout_passes=False)


@functools.lru_cache(maxsize=None)
def _make_sc_edge(e_pad, n_pad, per_w):
  n_chunks = per_w // CHUNK
  n_pairs = n_chunks // 2
  rpt = n_pad // NS

  @functools.partial(
      pl.kernel,
      out_type=[
          jax.ShapeDtypeStruct((NW, n_pad), jnp.float32),
          jax.ShapeDtypeStruct((NC, n_pad, D), jnp.float32),
      ],
      mesh=_sc_mesh(),
      compiler_params=_SC_PARAMS,
      scratch_types=[
          pltpu.VMEM((2, CHUNK), jnp.int32),
          pltpu.VMEM((CHUNK, D), jnp.float32),
          pltpu.VMEM((CHUNK, D), jnp.float32),
          pltpu.VMEM((2, CHUNK), jnp.int32),
          pltpu.VMEM((CHUNK, D), jnp.float32),
          pltpu.VMEM((CHUNK, D), jnp.float32),
          pltpu.VMEM((D,), jnp.float32),
          pltpu.VMEM((n_pad,), jnp.float32),
          pltpu.VMEM_SHARED((n_pad, D), jnp.float32),
          pltpu.SemaphoreType.DMA,
          pltpu.SemaphoreType.DMA,
      ],
  )
  def sc_edge(xl_hbm, xr_hbm, idx_hbm, att_hbm,
              zero_nd_hbm,
              s_hbm, out_hbm,
              idx_a, xlr_a, xrr_a,
              idx_b, xlr_b, xrr_b,
              att_v, s_v, acc_sh, sem_a, sem_b):
    cid = lax.axis_index("c")
    sid = lax.axis_index("s")
    wid = cid * NS + sid
    pltpu.sync_copy(att_hbm, att_v)
    pltpu.sync_copy(zero_nd_hbm.at[pl.ds(sid * rpt, rpt)],
                    acc_sh.at[pl.ds(sid * rpt, rpt)])

    def zero_body(i, carry):
      s_v[pl.ds(i * L, L)] = jnp.zeros((L,), jnp.float32)
      return carry

    lax.fori_loop(0, n_pad // L, zero_body, 0)
    plsc.subcore_barrier()
    row16 = lax.iota(jnp.int32, L)
    att_q = [att_v[pl.ds(q * L, L)] for q in range(D // L)]

    A = (idx_a, xlr_a, xrr_a, sem_a)
    B = (idx_b, xlr_b, xrr_b, sem_b)

    def issue(ci, buf):
      idxb, xlb, xrb, sem = buf
      gci = wid * n_chunks + ci
      pltpu.sync_copy(idx_hbm.at[gci], idxb)
      pltpu.async_copy(xl_hbm.at[idxb.at[0]], xlb, sem)
      pltpu.async_copy(xr_hbm.at[idxb.at[1]], xrb, sem)

    def wait_gather(buf):
      idxb, xlb, xrb, sem = buf
      pltpu.make_async_copy(xl_hbm.at[idxb.at[0]], xlb, sem).wait()
      pltpu.make_async_copy(xr_hbm.at[idxb.at[1]], xrb, sem).wait()

    def process(buf):
      idxb, xlb, xrb, _ = buf
      dstb = idxb.at[1]

      def group_body(g, carry2):
        p16 = jnp.zeros((L,), jnp.float32)
        for u in range(L):
          j = g * L + u
          acc = None
          xs = []
          for q in range(D // L):
            a = xlb[j, pl.ds(q * L, L)]
            xs.append(a)
            t = a + xrb[j, pl.ds(q * L, L)]
            t = jnp.maximum(t, 0.2 * t)
            pr = att_q[q] * t
            acc = pr if acc is None else acc + pr
          pb = jnp.exp(jnp.full((L,), jnp.sum(acc), jnp.float32))
          p16 = jnp.where(row16 == u, pb, p16)
          for q in range(D // L):
            xlb[j, pl.ds(q * L, L)] = xs[q] * pb
        plsc.addupdate_scatter(s_v, [idxb[1, pl.ds(g * L, L)]], p16)
        return carry2

      lax.fori_loop(0, CHUNK // L, group_body, 0)
      pltpu.sync_copy(xlb, acc_sh.at[dstb], add=True)

    issue(0, A)

    def pair_body(i, carry):
      issue(2 * i + 1, B)
      wait_gather(A)
      process(A)

      @pl.when(i < n_pairs - 1)
      def _():
        issue(2 * i + 2, A)

      wait_gather(B)
      process(B)
      return carry

    lax.fori_loop(0, n_pairs, pair_body, 0)
    pltpu.sync_copy(s_v, s_hbm.at[wid])
    plsc.subcore_barrier()
    pltpu.sync_copy(acc_sh.at[pl.ds(sid * rpt, rpt)],
                    out_hbm.at[cid, pl.ds(sid * rpt, rpt)])

  return sc_edge


def kernel(edge_index, emb, Wl1, Wr1, att1, b1, Wl2, Wr2, att2, b2):
  n = emb.shape[0]
  e2 = edge_index.shape[1] + n
  n_pad = ((n + BN) // BN) * BN
  per_w = -(-e2 // (NW * 2 * CHUNK)) * 2 * CHUNK
  e_pad = per_w * NW

  loop = jnp.arange(n, dtype=jnp.int32)
  pad_e = e_pad - e2
  src = jnp.concatenate(
      [edge_index[0], loop, jnp.zeros((pad_e,), jnp.int32)])
  dst = jnp.concatenate(
      [edge_index[1], loop, jnp.full((pad_e,), n, jnp.int32)])
  idx_pk = jnp.stack([src.reshape(-1, CHUNK), dst.reshape(-1, CHUNK)], axis=1)
  emb_p = jnp.pad(emb, ((0, n_pad - n), (0, 0)))
  zero_nd = jnp.zeros((n_pad, D), jnp.float32)
  b1r = b1.reshape(1, D)
  b2r = b2.reshape(1, D)
  sc_edge = _make_sc_edge(e_pad, n_pad, per_w)

  xl1, xr1 = _mm2(emb_p, Wl1, Wr1)
  s1, acc1 = sc_edge(xl1, xr1, idx_pk, att1, zero_nd)
  h, xl2, xr2 = _comb_mm2(acc1, s1, b1r, emb_p, Wl2, Wr2)
  s2, acc2 = sc_edge(xl2, xr2, idx_pk, att2, zero_nd)
  y = _final(acc2, s2, b2r, h)

  y = y[:n]
  return (y[:N_USERS], y[N_USERS:])


# speedup vs baseline: 1.3003x; 1.0006x over previous
import functools

import jax
import jax.numpy as jnp
from jax import lax
from jax.experimental import pallas as pl
from jax.experimental.pallas import tpu as pltpu
from jax.experimental.pallas import tpu_sc as plsc

N_USERS = 6000
D = 128
NC = 2
NS = 16
L = 16
NW = NC * NS
CHUNK = 64
BN = 1024


def _mm2_body(x_ref, wl_ref, wr_ref, xl_ref, xr_ref):
  x = x_ref[...]
  xl_ref[...] = jnp.dot(x, wl_ref[...], preferred_element_type=jnp.float32)
  xr_ref[...] = jnp.dot(x, wr_ref[...], preferred_element_type=jnp.float32)


def _mm2(x, wl, wr):
  n = x.shape[0]
  return pl.pallas_call(
      _mm2_body,
      grid=(n // BN,),
      in_specs=[
          pl.BlockSpec((BN, D), lambda i: (i, 0)),
          pl.BlockSpec((D, D), lambda i: (0, 0)),
          pl.BlockSpec((D, D), lambda i: (0, 0)),
      ],
      out_specs=[
          pl.BlockSpec((BN, D), lambda i: (i, 0)),
          pl.BlockSpec((BN, D), lambda i: (i, 0)),
      ],
      out_shape=[jax.ShapeDtypeStruct((n, D), jnp.float32)] * 2,
  )(x, wl, wr)


def _comb_mm2_body(acc_ref, s_ref, b_ref, res_ref, wl_ref, wr_ref,
                   h_ref, xl_ref, xr_ref):
  s = jnp.sum(s_ref[...], axis=0) + 1e-16
  h = (acc_ref[0] + acc_ref[1]) / s[:, None] + b_ref[...] + res_ref[...]
  h = jnp.maximum(h, 0.0)
  h_ref[...] = h
  xl_ref[...] = jnp.dot(h, wl_ref[...], preferred_element_type=jnp.float32)
  xr_ref[...] = jnp.dot(h, wr_ref[...], preferred_element_type=jnp.float32)


def _comb_mm2(acc, s, b, res, wl, wr):
  n = res.shape[0]
  nw = s.shape[0]
  return pl.pallas_call(
      _comb_mm2_body,
      grid=(n // BN,),
      in_specs=[
          pl.BlockSpec((NC, BN, D), lambda i: (0, i, 0)),
          pl.BlockSpec((nw, BN), lambda i: (0, i)),
          pl.BlockSpec((1, D), lambda i: (0, 0)),
          pl.BlockSpec((BN, D), lambda i: (i, 0)),
          pl.BlockSpec((D, D), lambda i: (0, 0)),
          pl.BlockSpec((D, D), lambda i: (0, 0)),
      ],
      out_specs=[
          pl.BlockSpec((BN, D), lambda i: (i, 0)),
          pl.BlockSpec((BN, D), lambda i: (i, 0)),
          pl.BlockSpec((BN, D), lambda i: (i, 0)),
      ],
      out_shape=[jax.ShapeDtypeStruct((n, D), jnp.float32)] * 3,
  )(acc, s, b, res, wl, wr)


def _final_body(acc_ref, s_ref, b_ref, res_ref, y_ref):
  s = jnp.sum(s_ref[...], axis=0) + 1e-16
  y_ref[...] = (acc_ref[0] + acc_ref[1]) / s[:, None] + b_ref[...] + res_ref[...]


def _final(acc, s, b, res):
  n = res.shape[0]
  nw = s.shape[0]
  return pl.pallas_call(
      _final_body,
      grid=(n // BN,),
      in_specs=[
          pl.BlockSpec((NC, BN, D), lambda i: (0, i, 0)),
          pl.BlockSpec((nw, BN), lambda i: (0, i)),
          pl.BlockSpec((1, D), lambda i: (0, 0)),
          pl.BlockSpec((BN, D), lambda i: (i, 0)),
      ],
      out_specs=pl.BlockSpec((BN, D), lambda i: (i, 0)),
      out_shape=jax.ShapeDtypeStruct((n, D), jnp.float32),
  )(acc, s, b, res)


def _sc_mesh():
  return plsc.VectorSubcoreMesh(
      core_axis_name="c", subcore_axis_name="s", num_cores=NC, num_subcores=NS)


_SC_PARAMS = pltpu.CompilerParams(needs_layout_passes=False)


@functools.lru_cache(maxsize=None)
def _make_sc_edge(e_pad, n_pad, per_w):
  n_chunks = per_w // CHUNK
  n_pairs = n_chunks // 2
  rpt = n_pad // NS

  @functools.partial(
      pl.kernel,
      out_type=[
          jax.ShapeDtypeStruct((NW, n_pad), jnp.float32),
          jax.ShapeDtypeStruct((NC, n_pad, D), jnp.float32),
      ],
      mesh=_sc_mesh(),
      compiler_params=_SC_PARAMS,
      scratch_types=[
          pltpu.VMEM((2, CHUNK), jnp.int32),
          pltpu.VMEM((CHUNK, D), jnp.float32),
          pltpu.VMEM((CHUNK, D), jnp.float32),
          pltpu.VMEM((2, CHUNK), jnp.int32),
          pltpu.VMEM((CHUNK, D), jnp.float32),
          pltpu.VMEM((CHUNK, D), jnp.float32),
          pltpu.VMEM((D,), jnp.float32),
          pltpu.VMEM((n_pad,), jnp.float32),
          pltpu.VMEM_SHARED((n_pad, D), jnp.float32),
          pltpu.SemaphoreType.DMA,
          pltpu.SemaphoreType.DMA,
      ],
  )
  def sc_edge(xl_hbm, xr_hbm, idx_hbm, att_hbm,
              zero_nd_hbm,
              s_hbm, out_hbm,
              idx_a, xlr_a, xrr_a,
              idx_b, xlr_b, xrr_b,
              att_v, s_v, acc_sh, sem_a, sem_b):
    cid = lax.axis_index("c")
    sid = lax.axis_index("s")
    wid = cid * NS + sid
    pltpu.sync_copy(att_hbm, att_v)
    pltpu.sync_copy(zero_nd_hbm.at[pl.ds(sid * rpt, rpt)],
                    acc_sh.at[pl.ds(sid * rpt, rpt)])

    def zero_body(i, carry):
      s_v[pl.ds(i * L, L)] = jnp.zeros((L,), jnp.float32)
      return carry

    lax.fori_loop(0, n_pad // L, zero_body, 0)
    plsc.subcore_barrier()
    row16 = lax.iota(jnp.int32, L)
    att_q = [att_v[pl.ds(q * L, L)] for q in range(D // L)]

    A = (idx_a, xlr_a, xrr_a, sem_a)
    B = (idx_b, xlr_b, xrr_b, sem_b)

    def issue(ci, buf):
      idxb, xlb, xrb, sem = buf
      gci = wid * n_chunks + ci
      pltpu.sync_copy(idx_hbm.at[gci], idxb)
      pltpu.async_copy(xl_hbm.at[idxb.at[0]], xlb, sem)
      pltpu.async_copy(xr_hbm.at[idxb.at[1]], xrb, sem)

    def wait_gather(buf):
      idxb, xlb, xrb, sem = buf
      pltpu.make_async_copy(xl_hbm.at[idxb.at[0]], xlb, sem).wait()
      pltpu.make_async_copy(xr_hbm.at[idxb.at[1]], xrb, sem).wait()

    def process(buf):
      idxb, xlb, xrb, _ = buf
      dstb = idxb.at[1]

      def group_body(g, carry2):
        p16 = jnp.zeros((L,), jnp.float32)
        for u in range(L):
          j = g * L + u
          acc = None
          xs = []
          for q in range(D // L):
            a = xlb[j, pl.ds(q * L, L)]
            xs.append(a)
            t = a + xrb[j, pl.ds(q * L, L)]
            t = jnp.maximum(t, 0.2 * t)
            pr = att_q[q] * t
            acc = pr if acc is None else acc + pr
          pb = jnp.exp(jnp.full((L,), jnp.sum(acc), jnp.float32))
          p16 = jnp.where(row16 == u, pb, p16)
          for q in range(D // L):
            xlb[j, pl.ds(q * L, L)] = xs[q] * pb
        plsc.addupdate_scatter(s_v, [idxb[1, pl.ds(g * L, L)]], p16)
        return carry2

      lax.fori_loop(0, CHUNK // L, group_body, 0)
      pltpu.sync_copy(xlb, acc_sh.at[dstb], add=True)

    issue(0, A)

    def pair_body(i, carry):
      issue(2 * i + 1, B)
      wait_gather(A)
      process(A)

      @pl.when(i < n_pairs - 1)
      def _():
        issue(2 * i + 2, A)

      wait_gather(B)
      process(B)
      return carry

    lax.fori_loop(0, n_pairs, pair_body, 0)
    pltpu.sync_copy(s_v, s_hbm.at[wid])
    plsc.subcore_barrier()
    pltpu.sync_copy(acc_sh.at[pl.ds(sid * rpt, rpt)],
                    out_hbm.at[cid, pl.ds(sid * rpt, rpt)])

  return sc_edge


def kernel(edge_index, emb, Wl1, Wr1, att1, b1, Wl2, Wr2, att2, b2):
  n = emb.shape[0]
  e2 = edge_index.shape[1] + n
  n_pad = ((n + BN) // BN) * BN
  per_w = -(-e2 // (NW * 2 * CHUNK)) * 2 * CHUNK
  e_pad = per_w * NW

  loop = jnp.arange(n, dtype=jnp.int32)
  pad_e = e_pad - e2
  src = jnp.concatenate(
      [edge_index[0], loop, jnp.zeros((pad_e,), jnp.int32)])
  dst = jnp.concatenate(
      [edge_index[1], loop, jnp.full((pad_e,), n, jnp.int32)])
  idx_pk = jnp.stack([src.reshape(-1, CHUNK), dst.reshape(-1, CHUNK)], axis=1)
  emb_p = jnp.pad(emb, ((0, n_pad - n), (0, 0)))
  zero_nd = jnp.zeros((n_pad, D), jnp.float32)
  b1r = b1.reshape(1, D)
  b2r = b2.reshape(1, D)
  sc_edge = _make_sc_edge(e_pad, n_pad, per_w)

  xl1, xr1 = _mm2(emb_p, Wl1, Wr1)
  s1, acc1 = sc_edge(xl1, xr1, idx_pk, att1, zero_nd)
  h, xl2, xr2 = _comb_mm2(acc1, s1, b1r, emb_p, Wl2, Wr2)
  s2, acc2 = sc_edge(xl2, xr2, idx_pk, att2, zero_nd)
  y = _final(acc2, s2, b2r, h)

  y = y[:n]
  return (y[:N_USERS], y[N_USERS:])
